# Initial kernel scaffold; baseline (speedup 1.0000x reference)
import jax
import jax.numpy as jnp
from jax.experimental import pallas as pl


def kernel(rpn_probs, bbox_deltas, anchors):
    raise NotImplementedError("write your pallas kernel here")


# speedup vs baseline: 14.1797x; 14.1797x over previous
import functools

import jax
import jax.numpy as jnp
from jax.experimental import pallas as pl

A = 20000
LANES = 128
ROWS = 160
APAD = ROWS * LANES
NUM_OUT = 500
OUT_ROWS = 512
THRESH = 0.7
NEG_INF = float("-inf")


def _nms_body(scores_ref, anc_ref, del_ref, out_ref):
    sc0 = scores_ref[0]
    ay1 = anc_ref[0, 0]
    ax1 = anc_ref[0, 1]
    ay2 = anc_ref[0, 2]
    ax2 = anc_ref[0, 3]
    dy = del_ref[0, 0]
    dx = del_ref[0, 1]
    dh = del_ref[0, 2]
    dw = del_ref[0, 3]

    h = ay2 - ay1
    w = ax2 - ax1
    cy = ay1 + 0.5 * h
    cx = ax1 + 0.5 * w
    cy = cy + dy * h
    cx = cx + dx * w
    h = h * jnp.exp(dh)
    w = w * jnp.exp(dw)
    y1 = jnp.clip(cy - 0.5 * h, 0.0, 1.0)
    x1 = jnp.clip(cx - 0.5 * w, 0.0, 1.0)
    y2 = jnp.clip(cy + 0.5 * h, 0.0, 1.0)
    x2 = jnp.clip(cx + 0.5 * w, 0.0, 1.0)
    areas = (y2 - y1) * (x2 - x1)

    iota = (jax.lax.broadcasted_iota(jnp.int32, (ROWS, LANES), 0) * LANES
            + jax.lax.broadcasted_iota(jnp.int32, (ROWS, LANES), 1))

    def step(i, scores):
        m = jnp.max(scores)
        idx = jnp.min(jnp.where(scores == m, iota, APAD))
        valid = m > NEG_INF
        sel = iota == idx
        by1 = jnp.sum(jnp.where(sel, y1, 0.0))
        bx1 = jnp.sum(jnp.where(sel, x1, 0.0))
        by2 = jnp.sum(jnp.where(sel, y2, 0.0))
        bx2 = jnp.sum(jnp.where(sel, x2, 0.0))
        yy1 = jnp.maximum(by1, y1)
        xx1 = jnp.maximum(bx1, x1)
        yy2 = jnp.minimum(by2, y2)
        xx2 = jnp.minimum(bx2, x2)
        inter = jnp.maximum(yy2 - yy1, 0.0) * jnp.maximum(xx2 - xx1, 0.0)
        area_b = (by2 - by1) * (bx2 - bx1)
        union = area_b + areas - inter
        iou = inter / jnp.maximum(union, 1e-12)
        supp = (iou > THRESH) | sel
        new_scores = jnp.where(supp, NEG_INF, scores)
        row = jnp.concatenate(
            [by1.reshape(1, 1), bx1.reshape(1, 1),
             by2.reshape(1, 1), bx2.reshape(1, 1)], axis=1)
        row = jnp.where(valid, row, 0.0)
        out_ref[0, pl.ds(i, 1), :] = row
        return new_scores

    jax.lax.fori_loop(0, NUM_OUT, step, sc0)


@jax.jit
def kernel(rpn_probs, bbox_deltas, anchors):
    B = rpn_probs.shape[0]
    pad = APAD - A
    scores = jnp.pad(rpn_probs[:, :, 1], ((0, 0), (0, pad)),
                     constant_values=NEG_INF).reshape(B, ROWS, LANES)
    anc = jnp.pad(anchors, ((0, 0), (0, pad), (0, 0))).transpose(0, 2, 1)
    anc = anc.reshape(B, 4, ROWS, LANES)
    dlt = jnp.pad(bbox_deltas, ((0, 0), (0, pad), (0, 0))).transpose(0, 2, 1)
    dlt = dlt.reshape(B, 4, ROWS, LANES)

    out = pl.pallas_call(
        _nms_body,
        grid=(B,),
        in_specs=[
            pl.BlockSpec((1, ROWS, LANES), lambda b: (b, 0, 0)),
            pl.BlockSpec((1, 4, ROWS, LANES), lambda b: (b, 0, 0, 0)),
            pl.BlockSpec((1, 4, ROWS, LANES), lambda b: (b, 0, 0, 0)),
        ],
        out_specs=pl.BlockSpec((1, OUT_ROWS, 4), lambda b: (b, 0, 0)),
        out_shape=jax.ShapeDtypeStruct((B, OUT_ROWS, 4), jnp.float32),
    )(scores, anc, dlt)
    return out[:, :NUM_OUT, :]


# speedup vs baseline: 14.6774x; 1.0351x over previous
import jax
import jax.numpy as jnp
from jax.experimental import pallas as pl
from jax.experimental.pallas import tpu as pltpu

A = 20000
LANES = 128
ROWS = 160
APAD = ROWS * LANES
NUM_OUT = 500
OUT_ROWS = 512
THRESH = 0.7
NEG_INF = float("-inf")
NB = 2


def _nms_body(scores_in, anc_ref, del_ref, out_ref, box_ref, sc_ref):
    for b in range(NB):
        ay1 = anc_ref[b, 0]
        ax1 = anc_ref[b, 1]
        ay2 = anc_ref[b, 2]
        ax2 = anc_ref[b, 3]
        h = ay2 - ay1
        w = ax2 - ax1
        cy = ay1 + 0.5 * h
        cx = ax1 + 0.5 * w
        cy = cy + del_ref[b, 0] * h
        cx = cx + del_ref[b, 1] * w
        h = h * jnp.exp(del_ref[b, 2])
        w = w * jnp.exp(del_ref[b, 3])
        y1 = jnp.clip(cy - 0.5 * h, 0.0, 1.0)
        x1 = jnp.clip(cx - 0.5 * w, 0.0, 1.0)
        y2 = jnp.clip(cy + 0.5 * h, 0.0, 1.0)
        x2 = jnp.clip(cx + 0.5 * w, 0.0, 1.0)
        box_ref[b, 0] = y1
        box_ref[b, 1] = x1
        box_ref[b, 2] = y2
        box_ref[b, 3] = x2
        box_ref[b, 4] = (y2 - y1) * (x2 - x1)
        sc_ref[b] = scores_in[b]

    iota2d = (jax.lax.broadcasted_iota(jnp.int32, (ROWS, LANES), 0) * LANES
              + jax.lax.broadcasted_iota(jnp.int32, (ROWS, LANES), 1))
    lane_iota = jax.lax.broadcasted_iota(jnp.int32, (1, LANES), 1)

    def step(i, carry):
        for b in range(NB):
            scores = sc_ref[b]
            m = jnp.max(scores)
            idx = jnp.min(jnp.where(scores == m, iota2d, APAD))
            valid = m > NEG_INF
            r = idx // LANES
            c = idx % LANES
            lm = lane_iota == c
            by1 = jnp.sum(jnp.where(lm, box_ref[b, 0, pl.ds(r, 1), :], 0.0))
            bx1 = jnp.sum(jnp.where(lm, box_ref[b, 1, pl.ds(r, 1), :], 0.0))
            by2 = jnp.sum(jnp.where(lm, box_ref[b, 2, pl.ds(r, 1), :], 0.0))
            bx2 = jnp.sum(jnp.where(lm, box_ref[b, 3, pl.ds(r, 1), :], 0.0))
            yy1 = jnp.maximum(by1, box_ref[b, 0])
            xx1 = jnp.maximum(bx1, box_ref[b, 1])
            yy2 = jnp.minimum(by2, box_ref[b, 2])
            xx2 = jnp.minimum(bx2, box_ref[b, 3])
            inter = (jnp.maximum(yy2 - yy1, 0.0)
                     * jnp.maximum(xx2 - xx1, 0.0))
            area_b = (by2 - by1) * (bx2 - bx1)
            union = area_b + box_ref[b, 4] - inter
            iou = inter / jnp.maximum(union, 1e-12)
            sc_ref[b] = jnp.where(iou > THRESH, NEG_INF, scores)
            sc_ref[b, pl.ds(r, 1), :] = jnp.where(
                lm, NEG_INF, sc_ref[b, pl.ds(r, 1), :])
            row = jnp.concatenate(
                [by1.reshape(1, 1), bx1.reshape(1, 1),
                 by2.reshape(1, 1), bx2.reshape(1, 1)], axis=1)
            out_ref[b, pl.ds(i, 1), :] = jnp.where(valid, row, 0.0)
        return carry

    jax.lax.fori_loop(0, NUM_OUT, step, 0)


@jax.jit
def kernel(rpn_probs, bbox_deltas, anchors):
    B = rpn_probs.shape[0]
    pad = APAD - A
    scores = jnp.pad(rpn_probs[:, :, 1], ((0, 0), (0, pad)),
                     constant_values=NEG_INF).reshape(B, ROWS, LANES)
    anc = jnp.pad(anchors, ((0, 0), (0, pad), (0, 0))).transpose(0, 2, 1)
    anc = anc.reshape(B, 4, ROWS, LANES)
    dlt = jnp.pad(bbox_deltas, ((0, 0), (0, pad), (0, 0))).transpose(0, 2, 1)
    dlt = dlt.reshape(B, 4, ROWS, LANES)

    out = pl.pallas_call(
        _nms_body,
        out_shape=jax.ShapeDtypeStruct((B, OUT_ROWS, 4), jnp.float32),
        scratch_shapes=[
            pltpu.VMEM((NB, 5, ROWS, LANES), jnp.float32),
            pltpu.VMEM((NB, ROWS, LANES), jnp.float32),
        ],
    )(scores, anc, dlt)
    return out[:, :NUM_OUT, :]
